# initial kernel scaffold (unmeasured)
import jax
import jax.numpy as jnp
from jax import lax
from jax.experimental import pallas as pl
from jax.experimental.pallas import tpu as pltpu

N_DEV = 16
B_LOC = 2
SQ = 128
SKV = 128
H_LOC = 4
DH = 64
D_MODEL = 512
R = B_LOC * SQ


def _mask():
    row = lax.broadcasted_iota(jnp.int32, (R, R), 0)
    col = lax.broadcasted_iota(jnp.int32, (R, R), 1)
    rb = row // SQ
    cb = col // SKV
    qb = (row % SQ) // 64
    kb = (col % SKV) // 64
    sparse = (qb == kb) | (kb == 0) | ((qb + kb) % 3 == 0)
    return (rb == cb) & sparse


def kernel(x, Wq, K_ext, V_ext, Wo):
    def body(x_ref, wq_ref, k_hbm, v_hbm, wo_ref, out_ref,
             xg_ref, p_ref, rs_ref, k_ref, v_ref,
             ag_send, ag_recv, rs_send, rs_recv, local_sems):
        my = lax.axis_index("i")
        left = lax.rem(my - 1 + N_DEV, N_DEV)
        right = lax.rem(my + 1, N_DEV)

        barrier = pltpu.get_barrier_semaphore()
        for nbr in (left, right):
            pl.semaphore_signal(barrier, inc=1, device_id=(nbr,),
                                device_id_type=pl.DeviceIdType.MESH)
        pl.semaphore_wait(barrier, 2)

        kcp = pltpu.make_async_copy(
            k_hbm.at[:, :, pl.ds(my * H_LOC, H_LOC), :], k_ref, local_sems.at[0])
        vcp = pltpu.make_async_copy(
            v_hbm.at[:, :, pl.ds(my * H_LOC, H_LOC), :], v_ref, local_sems.at[1])
        kcp.start()
        vcp.start()

        xg_ref[0] = x_ref[:].reshape(R, D_MODEL)
        for h in range(N_DEV - 1):
            rdma = pltpu.make_async_remote_copy(
                src_ref=xg_ref.at[h],
                dst_ref=xg_ref.at[h + 1],
                send_sem=ag_send.at[h],
                recv_sem=ag_recv.at[h],
                device_id=(right,),
                device_id_type=pl.DeviceIdType.MESH,
            )
            rdma.start()
            rdma.wait()

        kcp.wait()
        vcp.wait()

        mask = _mask()
        neg = jnp.float32(-1e9)

        def compute_partial(r):
            j = lax.rem(my - r + N_DEV, N_DEV)
            q = jnp.dot(xg_ref[r], wq_ref[:],
                        preferred_element_type=jnp.float32)
            ctx = []
            for h in range(H_LOC):
                qh = q[:, h * DH:(h + 1) * DH]
                kh = k_ref[pl.ds(j * B_LOC, B_LOC), :, h, :].reshape(R, DH)
                vh = v_ref[pl.ds(j * B_LOC, B_LOC), :, h, :].reshape(R, DH)
                sc = lax.dot_general(
                    qh, kh, (((1,), (1,)), ((), ())),
                    preferred_element_type=jnp.float32) * 0.125
                sc = jnp.where(mask, sc, neg)
                m = jnp.max(sc, axis=1, keepdims=True)
                w = jnp.exp(sc - m)
                w = w / jnp.sum(w, axis=1, keepdims=True)
                ctx.append(jnp.dot(w, vh, preferred_element_type=jnp.float32))
            ctx = jnp.concatenate(ctx, axis=1)
            return jnp.dot(ctx, wo_ref[:],
                           preferred_element_type=jnp.float32)

        for r in range(N_DEV):
            p_ref[r] = compute_partial(r)

        for s in range(N_DEV - 1):
            if s > 0:
                p_ref[s + 1] = p_ref[s + 1] + rs_ref[s - 1]
            rdma = pltpu.make_async_remote_copy(
                src_ref=p_ref.at[s + 1],
                dst_ref=rs_ref.at[s],
                send_sem=rs_send.at[s],
                recv_sem=rs_recv.at[s],
                device_id=(right,),
                device_id_type=pl.DeviceIdType.MESH,
            )
            rdma.start()
            rdma.wait()

        out_ref[:] = (p_ref[0] + rs_ref[N_DEV - 2]).reshape(B_LOC, SQ, D_MODEL)

    return pl.pallas_call(
        body,
        out_shape=jax.ShapeDtypeStruct((B_LOC, SQ, D_MODEL), jnp.float32),
        in_specs=[
            pl.BlockSpec(memory_space=pltpu.VMEM),
            pl.BlockSpec(memory_space=pltpu.VMEM),
            pl.BlockSpec(memory_space=pltpu.ANY),
            pl.BlockSpec(memory_space=pltpu.ANY),
            pl.BlockSpec(memory_space=pltpu.VMEM),
        ],
        out_specs=pl.BlockSpec(memory_space=pltpu.VMEM),
        scratch_shapes=[
            pltpu.VMEM((N_DEV, R, D_MODEL), jnp.float32),
            pltpu.VMEM((N_DEV, R, D_MODEL), jnp.float32),
            pltpu.VMEM((N_DEV - 1, R, D_MODEL), jnp.float32),
            pltpu.VMEM((2 * N_DEV, SKV, H_LOC, DH), jnp.float32),
            pltpu.VMEM((2 * N_DEV, SKV, H_LOC, DH), jnp.float32),
            pltpu.SemaphoreType.DMA((N_DEV - 1,)),
            pltpu.SemaphoreType.DMA((N_DEV - 1,)),
            pltpu.SemaphoreType.DMA((N_DEV - 1,)),
            pltpu.SemaphoreType.DMA((N_DEV - 1,)),
            pltpu.SemaphoreType.DMA((2,)),
        ],
        compiler_params=pltpu.CompilerParams(collective_id=0),
    )(x, Wq, K_ext, V_ext, Wo)


# baseline (device time: 488729 ns/iter reference)
import jax
import jax.numpy as jnp
from jax import lax
from jax.experimental import pallas as pl
from jax.experimental.pallas import tpu as pltpu

N_DEV = 16
B_LOC = 2
SQ = 128
SKV = 128
H_LOC = 4
DH = 64
D_MODEL = 512
R = B_LOC * SQ


def _mask():
    row = lax.broadcasted_iota(jnp.int32, (R, R), 0)
    col = lax.broadcasted_iota(jnp.int32, (R, R), 1)
    rb = row // SQ
    cb = col // SKV
    qb = (row % SQ) // 64
    kb = (col % SKV) // 64
    sparse = (qb == kb) | (kb == 0) | ((qb + kb) % 3 == 0)
    return (rb == cb) & sparse


def kernel(x, Wq, K_ext, V_ext, Wo):
    def body(x_ref, wq_ref, k_hbm, v_hbm, wo_ref, out_ref,
             xg_ref, p_ref, rs_ref, k_ref, v_ref,
             ag_send, ag_recv, rs_send, rs_recv, local_sems):
        my = lax.axis_index("i")
        left = lax.rem(my - 1 + N_DEV, N_DEV)
        right = lax.rem(my + 1, N_DEV)

        barrier = pltpu.get_barrier_semaphore()
        for nbr in (left, right):
            pl.semaphore_signal(barrier, inc=1, device_id=(nbr,),
                                device_id_type=pl.DeviceIdType.MESH)
        pl.semaphore_wait(barrier, 2)

        kcp = pltpu.make_async_copy(
            k_hbm.at[:, :, pl.ds(my * H_LOC, H_LOC), :], k_ref, local_sems.at[0])
        vcp = pltpu.make_async_copy(
            v_hbm.at[:, :, pl.ds(my * H_LOC, H_LOC), :], v_ref, local_sems.at[1])
        kcp.start()
        vcp.start()

        xg_ref[0] = x_ref[:].reshape(R, D_MODEL)
        for h in range(N_DEV - 1):
            rdma = pltpu.make_async_remote_copy(
                src_ref=xg_ref.at[h],
                dst_ref=xg_ref.at[h + 1],
                send_sem=ag_send.at[h],
                recv_sem=ag_recv.at[h],
                device_id=(right,),
                device_id_type=pl.DeviceIdType.MESH,
            )
            rdma.start()
            rdma.wait()

        kcp.wait()
        vcp.wait()

        mask = _mask()
        neg = jnp.float32(-1e9)

        def compute_partial(r):
            j = lax.rem(my - r + N_DEV, N_DEV)
            q = jnp.dot(xg_ref[r], wq_ref[:],
                        preferred_element_type=jnp.float32)
            ctx = []
            for h in range(H_LOC):
                qh = q[:, h * DH:(h + 1) * DH]
                kh = k_ref[pl.ds(j * B_LOC, B_LOC), :, h, :].reshape(R, DH)
                vh = v_ref[pl.ds(j * B_LOC, B_LOC), :, h, :].reshape(R, DH)
                sc = lax.dot_general(
                    qh, kh, (((1,), (1,)), ((), ())),
                    preferred_element_type=jnp.float32) * 0.125
                sc = jnp.where(mask, sc, neg)
                m = jnp.max(sc, axis=1, keepdims=True)
                w = jnp.exp(sc - m)
                w = w / jnp.sum(w, axis=1, keepdims=True)
                ctx.append(jnp.dot(w, vh, preferred_element_type=jnp.float32))
            ctx = jnp.concatenate(ctx, axis=1)
            return jnp.dot(ctx, wo_ref[:],
                           preferred_element_type=jnp.float32)

        for r in range(N_DEV):
            p_ref[r] = compute_partial(r)

        for s in range(N_DEV - 1):
            if s > 0:
                p_ref[s + 1] = p_ref[s + 1] + rs_ref[s - 1]
            rdma = pltpu.make_async_remote_copy(
                src_ref=p_ref.at[s + 1],
                dst_ref=rs_ref.at[s],
                send_sem=rs_send.at[s],
                recv_sem=rs_recv.at[s],
                device_id=(right,),
                device_id_type=pl.DeviceIdType.MESH,
            )
            rdma.start()
            rdma.wait()

        out_ref[:] = (p_ref[0] + rs_ref[N_DEV - 2]).reshape(B_LOC, SQ, D_MODEL)

    return pl.pallas_call(
        body,
        out_shape=jax.ShapeDtypeStruct((B_LOC, SQ, D_MODEL), jnp.float32),
        in_specs=[
            pl.BlockSpec(memory_space=pltpu.VMEM),
            pl.BlockSpec(memory_space=pltpu.VMEM),
            pl.BlockSpec(memory_space=pl.ANY),
            pl.BlockSpec(memory_space=pl.ANY),
            pl.BlockSpec(memory_space=pltpu.VMEM),
        ],
        out_specs=pl.BlockSpec(memory_space=pltpu.VMEM),
        scratch_shapes=[
            pltpu.VMEM((N_DEV, R, D_MODEL), jnp.float32),
            pltpu.VMEM((N_DEV, R, D_MODEL), jnp.float32),
            pltpu.VMEM((N_DEV - 1, R, D_MODEL), jnp.float32),
            pltpu.VMEM((2 * N_DEV, SKV, H_LOC, DH), jnp.float32),
            pltpu.VMEM((2 * N_DEV, SKV, H_LOC, DH), jnp.float32),
            pltpu.SemaphoreType.DMA((N_DEV - 1,)),
            pltpu.SemaphoreType.DMA((N_DEV - 1,)),
            pltpu.SemaphoreType.DMA((N_DEV - 1,)),
            pltpu.SemaphoreType.DMA((N_DEV - 1,)),
            pltpu.SemaphoreType.DMA((2,)),
        ],
        compiler_params=pltpu.CompilerParams(
            collective_id=0,
            vmem_limit_bytes=100 * 1024 * 1024,
        ),
    )(x, Wq, K_ext, V_ext, Wo)


# device time: 364686 ns/iter; 1.3401x vs baseline; 1.3401x over previous
import jax
import jax.numpy as jnp
from jax import lax
from jax.experimental import pallas as pl
from jax.experimental.pallas import tpu as pltpu

N_DEV = 16
B_LOC = 2
SQ = 128
SKV = 128
H_GRP = 4
DH = 64
D_MODEL = 512
HD_GRP = H_GRP * DH
R = B_LOC * SQ
N_CW = 8
N_CCW = 7


def _mask():
    row = lax.broadcasted_iota(jnp.int32, (R, R), 0)
    col = lax.broadcasted_iota(jnp.int32, (R, R), 1)
    rb = row // SQ
    cb = col // SKV
    qb = (row % SQ) // 64
    kb = (col % SKV) // 64
    sparse = (qb == kb) | (kb == 0) | ((qb + kb) % 3 == 0)
    return (rb == cb) & sparse


def kernel(x, Wq, K_ext, V_ext, Wo):
    def body(x_ref, wq_ref, k_hbm, v_hbm, wo_ref, out_ref,
             wq_buf, wo_buf, kt_ref, vt_ref, acc_ref,
             wq_cw_s, wq_cw_r, wq_ccw_s, wq_ccw_r,
             wo_cw_s, wo_cw_r, wo_ccw_s, wo_ccw_r, local_sems):
        my = lax.axis_index("i")
        left = lax.rem(my - 1 + N_DEV, N_DEV)
        right = lax.rem(my + 1, N_DEV)

        barrier = pltpu.get_barrier_semaphore()
        for nbr in (left, right):
            pl.semaphore_signal(barrier, inc=1, device_id=(nbr,),
                                device_id_type=pl.DeviceIdType.MESH)
        pl.semaphore_wait(barrier, 2)

        b0 = my * B_LOC
        copies = []
        for h in range(64):
            copies.append(pltpu.make_async_copy(
                k_hbm.at[pl.ds(b0, B_LOC), :, h, :], kt_ref.at[h],
                local_sems.at[0]))
            copies.append(pltpu.make_async_copy(
                v_hbm.at[pl.ds(b0, B_LOC), :, h, :], vt_ref.at[h],
                local_sems.at[1]))
        for cp in copies:
            cp.start()

        wq_buf[0] = wq_ref[:] * 0.125
        wo_buf[0] = wo_ref[:]

        for cp in copies:
            cp.wait()

        xl = x_ref[:].reshape(R, D_MODEL)
        maskf = _mask().astype(jnp.float32)

        def group_contrib(r):
            j = lax.rem(my - r + N_DEV, N_DEV)
            q = jnp.dot(xl, wq_buf[r],
                        preferred_element_type=jnp.float32)
            out = None
            for h in range(H_GRP):
                qh = q[:, h * DH:(h + 1) * DH]
                kh = kt_ref[j * H_GRP + h].reshape(R, DH)
                vh = vt_ref[j * H_GRP + h].reshape(R, DH)
                sc = lax.dot_general(
                    qh, kh, (((1,), (1,)), ((), ())),
                    preferred_element_type=jnp.float32)
                w = jnp.exp(sc) * maskf
                recip = 1.0 / jnp.sum(w, axis=1, keepdims=True)
                ctx = jnp.dot(w, vh,
                              preferred_element_type=jnp.float32) * recip
                contrib = jnp.dot(ctx, wo_buf[r, h * DH:(h + 1) * DH, :],
                                  preferred_element_type=jnp.float32)
                out = contrib if out is None else out + contrib
            return out

        def rdma(buf, src_slot, dst_slot, ssem, rsem, hop, dev):
            cp = pltpu.make_async_remote_copy(
                src_ref=buf.at[src_slot], dst_ref=buf.at[dst_slot],
                send_sem=ssem.at[hop], recv_sem=rsem.at[hop],
                device_id=(dev,), device_id_type=pl.DeviceIdType.MESH)
            cp.start()
            return cp

        for h in range(N_CW):
            cps = [
                rdma(wq_buf, h, h + 1, wq_cw_s, wq_cw_r, h, right),
                rdma(wo_buf, h, h + 1, wo_cw_s, wo_cw_r, h, right),
            ]
            if h < N_CCW:
                cps.append(rdma(wq_buf, (16 - h) % 16, 15 - h,
                                wq_ccw_s, wq_ccw_r, h, left))
                cps.append(rdma(wo_buf, (16 - h) % 16, 15 - h,
                                wo_ccw_s, wo_ccw_r, h, left))
            if h == 0:
                acc_ref[:] = group_contrib(0)
            else:
                acc_ref[:] = acc_ref[:] + group_contrib(h)
                acc_ref[:] = acc_ref[:] + group_contrib(16 - h)
            for cp in cps:
                cp.wait()
        acc_ref[:] = acc_ref[:] + group_contrib(8)

        out_ref[:] = acc_ref[:].reshape(B_LOC, SQ, D_MODEL)

    return pl.pallas_call(
        body,
        out_shape=jax.ShapeDtypeStruct((B_LOC, SQ, D_MODEL), jnp.float32),
        in_specs=[
            pl.BlockSpec(memory_space=pltpu.VMEM),
            pl.BlockSpec(memory_space=pltpu.VMEM),
            pl.BlockSpec(memory_space=pl.ANY),
            pl.BlockSpec(memory_space=pl.ANY),
            pl.BlockSpec(memory_space=pltpu.VMEM),
        ],
        out_specs=pl.BlockSpec(memory_space=pltpu.VMEM),
        scratch_shapes=[
            pltpu.VMEM((N_DEV, D_MODEL, HD_GRP), jnp.float32),
            pltpu.VMEM((N_DEV, HD_GRP, D_MODEL), jnp.float32),
            pltpu.VMEM((64, B_LOC, SKV, DH), jnp.float32),
            pltpu.VMEM((64, B_LOC, SKV, DH), jnp.float32),
            pltpu.VMEM((R, D_MODEL), jnp.float32),
            pltpu.SemaphoreType.DMA((N_CW,)),
            pltpu.SemaphoreType.DMA((N_CW,)),
            pltpu.SemaphoreType.DMA((N_CCW,)),
            pltpu.SemaphoreType.DMA((N_CCW,)),
            pltpu.SemaphoreType.DMA((N_CW,)),
            pltpu.SemaphoreType.DMA((N_CW,)),
            pltpu.SemaphoreType.DMA((N_CCW,)),
            pltpu.SemaphoreType.DMA((N_CCW,)),
            pltpu.SemaphoreType.DMA((2,)),
        ],
        compiler_params=pltpu.CompilerParams(
            collective_id=0,
            vmem_limit_bytes=100 * 1024 * 1024,
        ),
    )(x, Wq, K_ext, V_ext, Wo)
